# baseline (speedup 1.0000x reference)
import functools

import jax
import jax.numpy as jnp
from jax import lax
from jax.experimental import pallas as pl
from jax.experimental.pallas import tpu as pltpu
from jax.experimental.pallas import tpu_sc as plsc

_CHUNK = 32768
_NBUF = 3


def _sc_body(b, n, d, x_hbm, mask_hbm, start_hbm, end_hbm, out_hbm,
             buf0, buf1, buf2, maskbuf, zbuf, sem_in, sem_out, sem_row):
    bufs = [buf0, buf1, buf2]
    c = lax.axis_index("c")
    s = lax.axis_index("s")
    bpc = b // 2
    npc = 16 // bpc
    batch = c * bpc + s // npc
    sl = s % npc
    rows = n // npc
    chunk_count = (rows * d) // _CHUNK
    x_off = (batch * n + sl * rows) * d
    o_off = (batch * (n + 2) + 1 + sl * rows) * d
    ob_off = batch * (n + 2) * d

    is_leader = sl == 0
    is_zero_writer = sl == 1

    cpm = pltpu.make_async_copy(
        mask_hbm.at[pl.ds(batch * n, n)], maskbuf, sem_row)
    cpm.start()

    row_cps = []
    @pl.when(is_leader)
    def _():
        cp = pltpu.make_async_copy(
            start_hbm, out_hbm.at[pl.ds(ob_off, d)], sem_row)
        cp.start()
        row_cps.append(cp)

    cpm.wait()
    iota16 = lax.iota(jnp.int32, 16)

    def mbody(k, mv):
        v = maskbuf[pl.ds(k * 16, 16)]
        return jnp.minimum(mv, jnp.where(v != 0, n, iota16 + k * 16))

    mv = lax.fori_loop(0, n // 16, mbody, jnp.full((16,), n, jnp.int32))
    maskbuf[pl.ds(0, 16)] = mv
    fp = jnp.int32(n)
    for k in range(16):
        v = maskbuf[pl.ds(k, 16)]
        fp = jnp.minimum(fp, v[0])

    cps_in = [None] * chunk_count
    cps_out = [None] * chunk_count

    def start_in(j):
        cps_in[j] = pltpu.make_async_copy(
            x_hbm.at[pl.ds(x_off + j * _CHUNK, _CHUNK)],
            bufs[j % _NBUF], sem_in)
        cps_in[j].start()

    for j in range(min(_NBUF, chunk_count)):
        start_in(j)
    for j in range(chunk_count):
        cps_in[j].wait()
        co = pltpu.make_async_copy(
            bufs[j % _NBUF],
            out_hbm.at[pl.ds(o_off + j * _CHUNK, _CHUNK)], sem_out)
        co.start()
        cps_out[j] = co
        nj = j + _NBUF
        if nj < chunk_count:
            cps_out[j].wait()
            start_in(nj)
    for j in range(max(chunk_count - _NBUF, 0), chunk_count):
        cps_out[j].wait()

    @pl.when(is_zero_writer)
    def _():
        zv = jnp.zeros((16,), jnp.float32)

        def zbody(k, carry):
            zbuf[pl.ds(k * 16, 16)] = zv
            return carry

        lax.fori_loop(0, d // 16, zbody, 0)
        cp = pltpu.make_async_copy(
            zbuf, out_hbm.at[pl.ds(ob_off + (n + 1) * d, d)], sem_row)
        cp.start()
        cp.wait()

    @pl.when(is_leader)
    def _():
        row_cps[0].wait()

    plsc.subcore_barrier()

    @pl.when(is_leader)
    def _():
        cp = pltpu.make_async_copy(
            end_hbm, out_hbm.at[pl.ds(ob_off + fp * d, d)], sem_row)
        cp.start()
        cp.wait()


def kernel(x, protein_mask, start, end):
    b, n, d = x.shape
    mask_i32 = protein_mask.astype(jnp.int32)

    sc_call = pl.kernel(
        functools.partial(_sc_body, b, n, d),
        out_type=jax.ShapeDtypeStruct((b * (n + 2) * d,), jnp.float32),
        mesh=plsc.VectorSubcoreMesh(core_axis_name="c", subcore_axis_name="s"),
        scratch_types=[
            pltpu.VMEM((_CHUNK,), jnp.float32),
            pltpu.VMEM((_CHUNK,), jnp.float32),
            pltpu.VMEM((_CHUNK,), jnp.float32),
            pltpu.VMEM((n,), jnp.int32),
            pltpu.VMEM((d,), jnp.float32),
            pltpu.SemaphoreType.DMA,
            pltpu.SemaphoreType.DMA,
            pltpu.SemaphoreType.DMA,
        ],
    )
    out_flat = sc_call(x.reshape(-1), mask_i32.reshape(-1), start, end)
    return out_flat.reshape(b, n + 2, d)
